# baseline (speedup 1.0000x reference)
import functools
import math

import jax
import jax.numpy as jnp
from jax import lax
from jax.experimental import pallas as pl
from jax.experimental.pallas import tpu as pltpu
from jax.experimental.pallas import tpu_sc as plsc

_VOCAB = 100000
_CTX_DIM = 32
_T2V_DIM = 8
_D = 128
_B = 1024
_T = 200

_TCH = (2, 21, 51, 60, 66)
_R = 3 * _B


def _pick_chunk(n_per_w):
    for c in range(224, 0, -8):
        if n_per_w % c == 0:
            return c
    return n_per_w


def _sc_gather(table, idx_flat, off, n):
    info = plsc.get_sparse_core_info()
    nc, ns = info.num_cores, info.num_subcores
    nw = nc * ns
    d = table.shape[1]
    n_per_w = n // nw
    ch = _pick_chunk(n_per_w)
    n_ch = n_per_w // ch
    mesh = plsc.VectorSubcoreMesh(core_axis_name="c", subcore_axis_name="s")

    @functools.partial(
        pl.kernel,
        mesh=mesh,
        out_type=jax.ShapeDtypeStruct((n, d // 2), jnp.int32),
        scratch_types=[
            pltpu.VMEM((ch,), jnp.int32),
            pltpu.VMEM((ch,), jnp.int32),
            pltpu.VMEM((ch, d), jnp.float32),
            pltpu.VMEM((ch, d), jnp.float32),
            pltpu.VMEM((ch, d // 2), jnp.int32),
            pltpu.VMEM((ch, d // 2), jnp.int32),
            pltpu.SemaphoreType.DMA,
            pltpu.SemaphoreType.DMA,
        ],
    )
    def k(table_hbm, idx_hbm, out_hbm, idx_v0, idx_v1, rows_v0, rows_v1,
          pk_v0, pk_v1, sem0, sem1):
        wid = lax.axis_index("s") * nc + lax.axis_index("c")
        base = wid * n_per_w
        idx_bufs = (idx_v0, idx_v1)
        row_bufs = (rows_v0, rows_v1)
        pk_bufs = (pk_v0, pk_v1)
        sem_bufs = (sem0, sem1)

        def to_pairs(rows, pk):
            def body(r, carry):
                for g in range(d // 32):
                    a = lax.bitcast_convert_type(
                        rows[r, pl.ds(32 * g, 16)], jnp.int32)
                    b2 = lax.bitcast_convert_type(
                        rows[r, pl.ds(32 * g + 16, 16)], jnp.int32)
                    lo = lax.shift_right_logical(a + 0x8000, 16)
                    hi = (b2 + 0x8000) & jnp.int32(-65536)
                    pk[r, pl.ds(16 * g, 16)] = lo | hi
                return carry

            lax.fori_loop(0, ch, body, 0)
        pltpu.sync_copy(idx_hbm.at[pl.ds(off + base, ch)], idx_bufs[0])
        copies = [pltpu.async_copy(
            table_hbm.at[idx_bufs[0]], row_bufs[0], sem_bufs[0])]
        for g in range(n_ch):
            b = g % 2
            if g + 1 < n_ch:
                nxt = 1 - b
                pltpu.sync_copy(
                    idx_hbm.at[pl.ds(off + base + (g + 1) * ch, ch)],
                    idx_bufs[nxt])
                copies.append(pltpu.async_copy(
                    table_hbm.at[idx_bufs[nxt]], row_bufs[nxt], sem_bufs[nxt]))
            copies[g].wait()
            to_pairs(row_bufs[b], pk_bufs[b])
            pltpu.sync_copy(pk_bufs[b],
                            out_hbm.at[pl.ds(base + g * ch, ch)])

    return k(table, idx_flat)


def _unperm_matrix():
    rows = lax.broadcasted_iota(jnp.int32, (_D, _D), 0)
    cols = lax.broadcasted_iota(jnp.int32, (_D, _D), 1)
    g, m = cols // 32, cols % 32
    src2 = jnp.where(m < 16, 16 * g + m, 64 + 16 * g + (m - 16))
    return (rows == src2).astype(jnp.float32)


def _decode_pairs(xw):
    lo = lax.bitcast_convert_type(xw << 16, jnp.float32)
    hi = lax.bitcast_convert_type(xw & jnp.int32(-65536), jnp.float32)
    return jnp.concatenate([lo, hi], axis=1)


def _layernorm(x, gamma, beta):
    jmat = jnp.full((_D, _D), 1.0 / _D, dtype=jnp.float32)
    m1 = jnp.dot(x, jmat, preferred_element_type=jnp.float32)
    m2 = jnp.dot(x * x, jmat, preferred_element_type=jnp.float32)
    var = m2 - m1 * m1
    y = (x - m1) * lax.rsqrt(var + 1e-5) * gamma + beta
    return jnp.dot(y, _unperm_matrix(), preferred_element_type=jnp.float32)


def _time_vec(td_ref, fw_ref, fb_ref, wt_ref):
    n = td_ref.shape[2]
    args8 = jnp.broadcast_to(td_ref[0], (_T2V_DIM, n)) * fw_ref[:] + fb_ref[:]
    s8 = jnp.sin(args8)
    rowmask = lax.broadcasted_iota(jnp.int32, (_T2V_DIM, n), 0) == 0
    pt = jnp.where(rowmask, args8, s8)
    return lax.dot_general(
        pt, wt_ref[:], (((0,), (0,)), ((), ())),
        preferred_element_type=jnp.float32)


def _ev_body(tok_ref, td_ref, wt_ref, fw_ref, fb_ref, gamma_ref, beta_ref,
             out_ref):
    gamma = gamma_ref[:].reshape(1, _D)
    beta = beta_ref[:].reshape(1, _D)
    tv = _time_vec(td_ref, fw_ref, fb_ref, wt_ref)
    ev = (_decode_pairs(tok_ref[:]) + tv) * (1.0 / math.sqrt(_D))
    out_ref[:] = _layernorm(ev, gamma, beta)


def _head_body(tok_ref, td_ref, pc_ref, wt_ref, cw_ref, ctxtok_ref,
               fw_ref, fb_ref, gamma_ref, beta_ref, out_ref):
    i = pl.program_id(0)
    gamma = gamma_ref[:].reshape(1, _D)
    beta = beta_ref[:].reshape(1, _D)
    tv = _time_vec(td_ref, fw_ref, fb_ref, wt_ref)
    ev = (_decode_pairs(tok_ref[:]) + tv) * (1.0 / math.sqrt(_D))
    ctx = ctxtok_ref[:].reshape(1, _D) + jnp.dot(
        pc_ref[:], cw_ref[:], preferred_element_type=jnp.float32)
    x = jnp.where(i == 0, ctx, ev)
    out_ref[:] = _layernorm(x, gamma, beta)


_NROW = (_T + 1) * _B


def _small_specs():
    return [
        pl.BlockSpec((_T2V_DIM, _D), lambda i: (0, 0)),
        pl.BlockSpec((_T2V_DIM, 1), lambda i: (0, 0)),
        pl.BlockSpec((_T2V_DIM, 1), lambda i: (0, 0)),
        pl.BlockSpec((_D,), lambda i: (0,)),
        pl.BlockSpec((_D,), lambda i: (0,)),
    ]


def _head_chunk(tok, td, pc, wt, cw, ctok, fw, fb, gamma, beta, nsteps):
    grid = (nsteps,)
    in_specs = [
        pl.BlockSpec((_B, _D // 2), lambda i: (jnp.maximum(i - 1, 0), 0)),
        pl.BlockSpec((1, 1, _B), lambda i: (i, 0, 0)),
        pl.BlockSpec((_B, _CTX_DIM), lambda i: (0, 0)),
        pl.BlockSpec((_T2V_DIM, _D), lambda i: (0, 0)),
        pl.BlockSpec((_CTX_DIM, _D), lambda i: (0, 0)),
        pl.BlockSpec((_D,), lambda i: (0,)),
        pl.BlockSpec((_T2V_DIM, 1), lambda i: (0, 0)),
        pl.BlockSpec((_T2V_DIM, 1), lambda i: (0, 0)),
        pl.BlockSpec((_D,), lambda i: (0,)),
        pl.BlockSpec((_D,), lambda i: (0,)),
    ]
    return pl.pallas_call(
        _head_body,
        grid=grid,
        in_specs=in_specs,
        out_specs=pl.BlockSpec((_B, _D), lambda i: (i, 0)),
        out_shape=jax.ShapeDtypeStruct((_NROW, _D), jnp.float32),
    )(tok, td, pc, wt, cw, ctok, fw, fb, gamma, beta)


def _ev_chunk(out_prev, tok, td, weights, r0, nsteps):
    def out_map(i, r0=r0):
        return (r0 + i, 0)

    def td_map(i, r0=r0):
        return (r0 + i, 0, 0)

    return pl.pallas_call(
        lambda prev_ref, *refs: _ev_body(*refs),
        grid=(nsteps,),
        in_specs=[
            pl.BlockSpec(memory_space=pl.ANY),
            pl.BlockSpec((_R, _D // 2), lambda i: (i, 0)),
            pl.BlockSpec((1, 1, _R), td_map),
        ] + _small_specs(),
        out_specs=pl.BlockSpec((_R, _D), out_map),
        out_shape=jax.ShapeDtypeStruct((_NROW, _D), jnp.float32),
        input_output_aliases={0: 0},
    )(out_prev, tok, td, *weights)


def kernel(token_ids, time_deltas, patient_contexts, token_table, t2v_lin_w,
           t2v_lin_b, t2v_freq_w, t2v_freq_b, time_proj_w, ctx_token,
           context_proj_w, ln_gamma, ln_beta):
    idx_tm = token_ids.T.reshape(-1)
    td_tm = time_deltas.T.reshape(-1)

    perm = [32 * (j // 16) + (j % 16) for j in range(64)] \
        + [32 * (j // 16) + 16 + (j % 16) for j in range(64)]
    perm = jnp.array(perm, dtype=jnp.int32)
    wt = jnp.take(time_proj_w.T, perm, axis=1)
    cw = jnp.take(context_proj_w.T, perm, axis=1)
    ctx_token = jnp.take(ctx_token, perm, axis=0)
    gamma_p = jnp.take(ln_gamma, perm, axis=0)
    beta_p = jnp.take(ln_beta, perm, axis=0)
    fw = jnp.concatenate([t2v_lin_w.reshape(1), t2v_freq_w.reshape(-1)],
                         axis=0).reshape(_T2V_DIM, 1)
    fb = jnp.concatenate([t2v_lin_b, t2v_freq_b], axis=0).reshape(_T2V_DIM, 1)
    weights = (wt, fw, fb, gamma_p, beta_p)

    t0s = [sum(_TCH[:s]) for s in range(len(_TCH))]
    toks = [_sc_gather(token_table, idx_tm, t0 * _B, tc * _B)
            for t0, tc in zip(t0s, _TCH)]

    td_pad = jnp.concatenate([jnp.zeros((_B,), jnp.float32), td_tm])
    td_head = td_pad[:(1 + _TCH[0]) * _B].reshape(-1, 1, _B)
    td_ev = td_pad.reshape(-1, 1, _R)

    out = _head_chunk(toks[0], td_head, patient_contexts, wt, cw, ctx_token,
                      fw, fb, gamma_p, beta_p, nsteps=1 + _TCH[0])

    for s in range(1, len(_TCH)):
        t0, tc = t0s[s], _TCH[s]
        out = _ev_chunk(out, toks[s], td_ev, weights,
                        r0=(1 + t0) * _B // _R, nsteps=tc * _B // _R)

    return out.reshape(_T + 1, _B, _D).swapaxes(0, 1)


# speedup vs baseline: 1.0390x; 1.0390x over previous
import functools
import math

import jax
import jax.numpy as jnp
from jax import lax
from jax.experimental import pallas as pl
from jax.experimental.pallas import tpu as pltpu
from jax.experimental.pallas import tpu_sc as plsc

_VOCAB = 100000
_CTX_DIM = 32
_T2V_DIM = 8
_D = 128
_B = 1024
_T = 200

_TCH = (2, 15, 36, 48, 60, 39)
_R = 3 * _B


def _pick_chunk(n_per_w):
    for c in range(320, 0, -8):
        if n_per_w % c == 0:
            return c
    return n_per_w


def _sc_gather(table, idx_flat, off, n):
    info = plsc.get_sparse_core_info()
    nc, ns = info.num_cores, info.num_subcores
    nw = nc * ns
    d = table.shape[1]
    n_per_w = n // nw
    ch = _pick_chunk(n_per_w)
    n_ch = n_per_w // ch
    mesh = plsc.VectorSubcoreMesh(core_axis_name="c", subcore_axis_name="s")

    @functools.partial(
        pl.kernel,
        mesh=mesh,
        out_type=jax.ShapeDtypeStruct((n, d), jnp.float32),
        scratch_types=[
            pltpu.VMEM((ch,), jnp.int32),
            pltpu.VMEM((ch,), jnp.int32),
            pltpu.VMEM((ch, d), jnp.float32),
            pltpu.VMEM((ch, d), jnp.float32),
            pltpu.SemaphoreType.DMA,
            pltpu.SemaphoreType.DMA,
        ],
    )
    def k(table_hbm, idx_hbm, out_hbm, idx_v0, idx_v1, rows_v0, rows_v1,
          sem0, sem1):
        wid = lax.axis_index("s") * nc + lax.axis_index("c")
        base = wid * n_per_w
        idx_bufs = (idx_v0, idx_v1)
        row_bufs = (rows_v0, rows_v1)
        sem_bufs = (sem0, sem1)
        pltpu.sync_copy(idx_hbm.at[pl.ds(off + base, ch)], idx_bufs[0])
        copies = [pltpu.async_copy(
            table_hbm.at[idx_bufs[0]], row_bufs[0], sem_bufs[0])]
        for g in range(n_ch):
            b = g % 2
            if g + 1 < n_ch:
                nxt = 1 - b
                pltpu.sync_copy(
                    idx_hbm.at[pl.ds(off + base + (g + 1) * ch, ch)],
                    idx_bufs[nxt])
                copies.append(pltpu.async_copy(
                    table_hbm.at[idx_bufs[nxt]], row_bufs[nxt], sem_bufs[nxt]))
            copies[g].wait()
            pltpu.sync_copy(row_bufs[b],
                            out_hbm.at[pl.ds(base + g * ch, ch)])

    return k(table, idx_flat)


def _layernorm(x, gamma, beta):
    jmat = jnp.full((_D, _D), 1.0 / _D, dtype=jnp.float32)
    m1 = jnp.dot(x, jmat, preferred_element_type=jnp.float32)
    m2 = jnp.dot(x * x, jmat, preferred_element_type=jnp.float32)
    var = m2 - m1 * m1
    return (x - m1) * lax.rsqrt(var + 1e-5) * gamma + beta


def _time_vec(td_ref, fw_ref, fb_ref, wt_ref):
    n = td_ref.shape[2]
    args8 = jnp.broadcast_to(td_ref[0], (_T2V_DIM, n)) * fw_ref[:] + fb_ref[:]
    s8 = jnp.sin(args8)
    rowmask = lax.broadcasted_iota(jnp.int32, (_T2V_DIM, n), 0) == 0
    pt = jnp.where(rowmask, args8, s8)
    return lax.dot_general(
        pt, wt_ref[:], (((0,), (0,)), ((), ())),
        preferred_element_type=jnp.float32)


def _ev_body(tok_ref, td_ref, wt_ref, fw_ref, fb_ref, gamma_ref, beta_ref,
             out_ref):
    gamma = gamma_ref[:].reshape(1, _D)
    beta = beta_ref[:].reshape(1, _D)
    tv = _time_vec(td_ref, fw_ref, fb_ref, wt_ref)
    ev = (tok_ref[:] + tv) * (1.0 / math.sqrt(_D))
    out_ref[:] = _layernorm(ev, gamma, beta)


def _head_body(tok_ref, td_ref, pc_ref, wt_ref, cw_ref, ctxtok_ref,
               fw_ref, fb_ref, gamma_ref, beta_ref, out_ref):
    i = pl.program_id(0)
    gamma = gamma_ref[:].reshape(1, _D)
    beta = beta_ref[:].reshape(1, _D)
    tv = _time_vec(td_ref, fw_ref, fb_ref, wt_ref)
    ev = (tok_ref[:] + tv) * (1.0 / math.sqrt(_D))
    ctx = ctxtok_ref[:].reshape(1, _D) + jnp.dot(
        pc_ref[:], cw_ref[:], preferred_element_type=jnp.float32)
    x = jnp.where(i == 0, ctx, ev)
    out_ref[:] = _layernorm(x, gamma, beta)


_NROW = (_T + 1) * _B


def _small_specs():
    return [
        pl.BlockSpec((_T2V_DIM, _D), lambda i: (0, 0)),
        pl.BlockSpec((_T2V_DIM, 1), lambda i: (0, 0)),
        pl.BlockSpec((_T2V_DIM, 1), lambda i: (0, 0)),
        pl.BlockSpec((_D,), lambda i: (0,)),
        pl.BlockSpec((_D,), lambda i: (0,)),
    ]


def _head_chunk(tok, td, pc, wt, cw, ctok, fw, fb, gamma, beta, nsteps):
    grid = (nsteps,)
    in_specs = [
        pl.BlockSpec((_B, _D), lambda i: (jnp.maximum(i - 1, 0), 0)),
        pl.BlockSpec((1, 1, _B), lambda i: (i, 0, 0)),
        pl.BlockSpec((_B, _CTX_DIM), lambda i: (0, 0)),
        pl.BlockSpec((_T2V_DIM, _D), lambda i: (0, 0)),
        pl.BlockSpec((_CTX_DIM, _D), lambda i: (0, 0)),
        pl.BlockSpec((_D,), lambda i: (0,)),
        pl.BlockSpec((_T2V_DIM, 1), lambda i: (0, 0)),
        pl.BlockSpec((_T2V_DIM, 1), lambda i: (0, 0)),
        pl.BlockSpec((_D,), lambda i: (0,)),
        pl.BlockSpec((_D,), lambda i: (0,)),
    ]
    return pl.pallas_call(
        _head_body,
        grid=grid,
        in_specs=in_specs,
        out_specs=pl.BlockSpec((_B, _D), lambda i: (i, 0)),
        out_shape=jax.ShapeDtypeStruct((_NROW, _D), jnp.float32),
    )(tok, td, pc, wt, cw, ctok, fw, fb, gamma, beta)


def _ev_chunk(out_prev, tok, td, weights, r0, nsteps):
    def out_map(i, r0=r0):
        return (r0 + i, 0)

    def td_map(i, r0=r0):
        return (r0 + i, 0, 0)

    return pl.pallas_call(
        lambda prev_ref, *refs: _ev_body(*refs),
        grid=(nsteps,),
        in_specs=[
            pl.BlockSpec(memory_space=pl.ANY),
            pl.BlockSpec((_R, _D), lambda i: (i, 0)),
            pl.BlockSpec((1, 1, _R), td_map),
        ] + _small_specs(),
        out_specs=pl.BlockSpec((_R, _D), out_map),
        out_shape=jax.ShapeDtypeStruct((_NROW, _D), jnp.float32),
        input_output_aliases={0: 0},
    )(out_prev, tok, td, *weights)


def kernel(token_ids, time_deltas, patient_contexts, token_table, t2v_lin_w,
           t2v_lin_b, t2v_freq_w, t2v_freq_b, time_proj_w, ctx_token,
           context_proj_w, ln_gamma, ln_beta):
    idx_tm = token_ids.T.reshape(-1)
    td_tm = time_deltas.T.reshape(-1)

    wt = time_proj_w.T
    cw = context_proj_w.T
    fw = jnp.concatenate([t2v_lin_w.reshape(1), t2v_freq_w.reshape(-1)],
                         axis=0).reshape(_T2V_DIM, 1)
    fb = jnp.concatenate([t2v_lin_b, t2v_freq_b], axis=0).reshape(_T2V_DIM, 1)
    weights = (wt, fw, fb, ln_gamma, ln_beta)

    t0s = [sum(_TCH[:s]) for s in range(len(_TCH))]
    toks = [_sc_gather(token_table, idx_tm, t0 * _B, tc * _B)
            for t0, tc in zip(t0s, _TCH)]

    td_pad = jnp.concatenate([jnp.zeros((_B,), jnp.float32), td_tm])
    td_head = td_pad[:(1 + _TCH[0]) * _B].reshape(-1, 1, _B)
    td_ev = td_pad.reshape(-1, 1, _R)

    out = _head_chunk(toks[0], td_head, patient_contexts, wt, cw, ctx_token,
                      fw, fb, ln_gamma, ln_beta, nsteps=1 + _TCH[0])

    for s in range(1, len(_TCH)):
        t0, tc = t0s[s], _TCH[s]
        out = _ev_chunk(out, toks[s], td_ev, weights,
                        r0=(1 + t0) * _B // _R, nsteps=tc * _B // _R)

    return out.reshape(_T + 1, _B, _D).swapaxes(0, 1)
